# baseline (speedup 1.0000x reference)
import functools
import math

import jax
import jax.numpy as jnp
from jax import lax
from jax.experimental import pallas as pl
from jax.experimental.pallas import tpu as pltpu
from jax.experimental.pallas import tpu_sc as plsc

VOCAB = 1_000_000
D_MODEL = 64
SCALE = math.sqrt(D_MODEL)

NUM_CORES = 2
NUM_SUBCORES = 16
LANES = 16
NUM_WORKERS = NUM_CORES * NUM_SUBCORES

FULL_BLK = 7812
BLK_PER_W = 245

_MESH = dict(core_axis_name="c", subcore_axis_name="s",
             num_cores=NUM_CORES, num_subcores=NUM_SUBCORES)
_PARAMS = pltpu.CompilerParams(use_tc_tiling_on_sc=True,
                               needs_layout_passes=False)


def _worker_id():
  return lax.axis_index("s") * NUM_CORES + lax.axis_index("c")


@functools.partial(
    pl.kernel,
    out_type=jax.ShapeDtypeStruct((VOCAB, 128), jnp.float32),
    mesh=plsc.VectorSubcoreMesh(**_MESH),
    scratch_types=[
        pltpu.VMEM((2, 64, 128), jnp.float32),
        pltpu.VMEM((2, 128, 128), jnp.float32),
        pltpu.VMEM((16, 17), jnp.float32),
        pltpu.SemaphoreType.DMA((2,)),
        pltpu.SemaphoreType.DMA((2,)),
    ],
    compiler_params=_PARAMS,
)
def _convert(tt_hbm, tailw_hbm, tw_hbm, src, dst, stage, isem, osem):
  w = _worker_id()
  riota = lax.iota(jnp.int32, LANES)

  @pl.when(w == 0)
  def _tail():
    pltpu.sync_copy(tailw_hbm, src.at[0])
    pltpu.sync_copy(src.at[0], tw_hbm.at[pl.ds(FULL_BLK * 128, 64)])

  def transpose_block(b):
    @pl.loop(0, 8)
    def _jg(jg):
      for dg in range(4):
        for l in range(LANES):
          stage[l, pl.ds(0, LANES)] = src[b, 16 * dg + l, pl.ds(16 * jg, LANES)]
        for jj in range(LANES):
          col = jnp.full((LANES,), jj, jnp.int32)
          val = plsc.load_gather(stage, [riota, col])
          dst[b, 16 * jg + jj, pl.ds(16 * dg, LANES)] = val

  @pl.loop(0, BLK_PER_W + 1, step=2)
  def _group(t0):
    for b in range(2):
      c = w + NUM_WORKERS * (t0 + b)

      @pl.when(c < FULL_BLK)
      def _fire(b=b, c=c):
        pltpu.async_copy(
            tt_hbm.at[:, pl.ds(c * 128, 128)], src.at[b], isem.at[b])

    for b in range(2):
      c = w + NUM_WORKERS * (t0 + b)

      @pl.when(c < FULL_BLK)
      def _do(b=b, c=c):
        pltpu.make_async_copy(
            tt_hbm.at[:, pl.ds(c * 128, 128)], src.at[b], isem.at[b]).wait()
        transpose_block(b)
        pltpu.async_copy(
            dst.at[b], tw_hbm.at[pl.ds(c * 128, 128)], osem.at[b])

    for b in range(2):
      c = w + NUM_WORKERS * (t0 + b)

      @pl.when(c < FULL_BLK)
      def _drain(b=b, c=c):
        pltpu.make_async_copy(
            dst.at[b], tw_hbm.at[pl.ds(c * 128, 128)], osem.at[b]).wait()


COLS_PER_W = 6400 // NUM_WORKERS


@functools.partial(
    pl.kernel,
    out_type=jax.ShapeDtypeStruct((200, 8, 32, 8, 128), jnp.float32),
    mesh=plsc.VectorSubcoreMesh(**_MESH),
    scratch_types=[
        pltpu.VMEM((2, 128), jnp.int32),
        pltpu.VMEM((2, 128, 128), jnp.float32),
        pltpu.VMEM((2, 64, 128), jnp.float32),
        pltpu.VMEM((16, 65), jnp.float32),
        pltpu.SemaphoreType.DMA((2,)),
        pltpu.SemaphoreType.DMA((2,)),
        pltpu.SemaphoreType.DMA((2,)),
    ],
    compiler_params=_PARAMS,
)
def _lookup(tw_hbm, xt_hbm, o5_hbm, idx, rows, outb, stage,
            isem, gsem, osem):
  w = _worker_id()
  t_base = w * COLS_PER_W
  riota = lax.iota(jnp.int32, LANES)

  @pl.loop(0, COLS_PER_W, step=2)
  def _group(g0):
    hs, cs, idx_cps = [], [], []
    for b in range(2):
      t = t_base + g0 + b
      h = t >> 5
      c = t & 31
      hs.append(h)
      cs.append(c)
      idx_cps.append(pltpu.async_copy(
          xt_hbm.at[h, pl.ds(c * 128, 128)], idx.at[b], isem.at[b]))
    row_cps = []
    for b in range(2):
      idx_cps[b].wait()
      row_cps.append(pltpu.async_copy(
          tw_hbm.at[idx.at[b]], rows.at[b], gsem.at[b]))
    out_cps = []
    for b in range(2):
      row_cps[b].wait()

      @pl.loop(0, 8)
      def _jg(jg, b=b):
        for l in range(LANES):
          for md in range(4):
            stage[l, pl.ds(16 * md, LANES)] = (
                rows[b, 16 * jg + l, pl.ds(16 * md, LANES)])
        for d in range(D_MODEL):
          col = jnp.full((LANES,), d, jnp.int32)
          val = plsc.load_gather(stage, [riota, col])
          outb[b, d, pl.ds(16 * jg, LANES)] = val * SCALE

      for k in range(8):
        out_cps.append(pltpu.async_copy(
            outb.at[b].at[pl.ds(8 * k, 8)],
            o5_hbm.at[hs[b], k, cs[b]], osem.at[b]))
    for cp in out_cps:
      cp.wait()


def kernel(x, table):
  xt = x.astype(jnp.int32).T
  tt = table.T
  tail = table[FULL_BLK * 128:]
  tailw = jnp.concatenate([tail, tail], axis=1)
  tw = _convert(tt, tailw)
  o5 = _lookup(tw, xt)
  return o5.transpose(2, 4, 0, 1, 3).reshape(4096, 200, 64)


# speedup vs baseline: 1.4807x; 1.4807x over previous
import functools
import math

import jax
import jax.numpy as jnp
from jax import lax
from jax.experimental import pallas as pl
from jax.experimental.pallas import tpu as pltpu
from jax.experimental.pallas import tpu_sc as plsc

VOCAB = 1_000_000
D_MODEL = 64
SCALE = math.sqrt(D_MODEL)

NUM_CORES = 2
NUM_SUBCORES = 16
LANES = 16
NUM_WORKERS = NUM_CORES * NUM_SUBCORES

FULL_BLK = 7812
BLK_PER_W = 245

_MESH = dict(core_axis_name="c", subcore_axis_name="s",
             num_cores=NUM_CORES, num_subcores=NUM_SUBCORES)
_PARAMS = pltpu.CompilerParams(use_tc_tiling_on_sc=True,
                               needs_layout_passes=False)


def _worker_id():
  return lax.axis_index("s") * NUM_CORES + lax.axis_index("c")


@functools.partial(
    pl.kernel,
    out_type=jax.ShapeDtypeStruct((VOCAB, 128), jnp.float32),
    mesh=plsc.VectorSubcoreMesh(**_MESH),
    scratch_types=[
        pltpu.VMEM((2, 64, 128), jnp.float32),
        pltpu.VMEM((2, 128, 128), jnp.float32),
        pltpu.VMEM((8, 4, 16, 17), jnp.float32),
        pltpu.SemaphoreType.DMA((2,)),
        pltpu.SemaphoreType.DMA((2,)),
    ],
    compiler_params=_PARAMS,
)
def _convert(tt_hbm, tailw_hbm, tw_hbm, src, dst, stage, isem, osem):
  w = _worker_id()
  riota = lax.iota(jnp.int32, LANES)

  @pl.when(w == 0)
  def _tail():
    pltpu.sync_copy(tailw_hbm, src.at[0])
    pltpu.sync_copy(src.at[0], tw_hbm.at[pl.ds(FULL_BLK * 128, 64)])

  def transpose_block(b):
    @plsc.parallel_loop(0, 8, unroll=2)
    def _jg(jg):
      for dg in range(4):
        for l in range(LANES):
          stage[jg, dg, l, pl.ds(0, LANES)] = (
              src[b, 16 * dg + l, pl.ds(16 * jg, LANES)])
        for jj in range(LANES):
          col = jnp.full((LANES,), jj, jnp.int32)
          val = plsc.load_gather(stage.at[jg, dg], [riota, col])
          dst[b, 16 * jg + jj, pl.ds(16 * dg, LANES)] = val

  @pl.loop(0, BLK_PER_W + 1, step=2)
  def _group(t0):
    for b in range(2):
      c = w + NUM_WORKERS * (t0 + b)

      @pl.when(c < FULL_BLK)
      def _fire(b=b, c=c):
        pltpu.async_copy(
            tt_hbm.at[:, pl.ds(c * 128, 128)], src.at[b], isem.at[b])

    for b in range(2):
      c = w + NUM_WORKERS * (t0 + b)

      @pl.when(c < FULL_BLK)
      def _do(b=b, c=c):
        pltpu.make_async_copy(
            tt_hbm.at[:, pl.ds(c * 128, 128)], src.at[b], isem.at[b]).wait()
        transpose_block(b)
        pltpu.async_copy(
            dst.at[b], tw_hbm.at[pl.ds(c * 128, 128)], osem.at[b])

    for b in range(2):
      c = w + NUM_WORKERS * (t0 + b)

      @pl.when(c < FULL_BLK)
      def _drain(b=b, c=c):
        pltpu.make_async_copy(
            dst.at[b], tw_hbm.at[pl.ds(c * 128, 128)], osem.at[b]).wait()


COLS_PER_W = 6400 // NUM_WORKERS


@functools.partial(
    pl.kernel,
    out_type=jax.ShapeDtypeStruct((200, 8, 32, 8, 128), jnp.float32),
    mesh=plsc.VectorSubcoreMesh(**_MESH),
    scratch_types=[
        pltpu.VMEM((2, 128), jnp.int32),
        pltpu.VMEM((2, 128, 128), jnp.float32),
        pltpu.VMEM((2, 64, 128), jnp.float32),
        pltpu.VMEM((8, 16, 65), jnp.float32),
        pltpu.SemaphoreType.DMA((2,)),
        pltpu.SemaphoreType.DMA((2,)),
        pltpu.SemaphoreType.DMA((2,)),
    ],
    compiler_params=_PARAMS,
)
def _lookup(tw_hbm, xt_hbm, o5_hbm, idx, rows, outb, stage,
            isem, gsem, osem):
  w = _worker_id()
  t_base = w * COLS_PER_W
  riota = lax.iota(jnp.int32, LANES)

  @pl.loop(0, COLS_PER_W, step=2)
  def _group(g0):
    hs, cs, idx_cps = [], [], []
    for b in range(2):
      t = t_base + g0 + b
      h = t >> 5
      c = t & 31
      hs.append(h)
      cs.append(c)
      idx_cps.append(pltpu.async_copy(
          xt_hbm.at[h, pl.ds(c * 128, 128)], idx.at[b], isem.at[b]))
    row_cps = []
    for b in range(2):
      idx_cps[b].wait()
      row_cps.append(pltpu.async_copy(
          tw_hbm.at[idx.at[b]], rows.at[b], gsem.at[b]))
    out_cps = []
    for b in range(2):
      row_cps[b].wait()

      @plsc.parallel_loop(0, 8, unroll=2)
      def _jg(jg, b=b):
        for l in range(LANES):
          for md in range(4):
            stage[jg, l, pl.ds(16 * md, LANES)] = (
                rows[b, 16 * jg + l, pl.ds(16 * md, LANES)])
        for d in range(D_MODEL):
          col = jnp.full((LANES,), d, jnp.int32)
          val = plsc.load_gather(stage.at[jg], [riota, col])
          outb[b, d, pl.ds(16 * jg, LANES)] = val * SCALE

      for k in range(8):
        out_cps.append(pltpu.async_copy(
            outb.at[b].at[pl.ds(8 * k, 8)],
            o5_hbm.at[hs[b], k, cs[b]], osem.at[b]))
    for cp in out_cps:
      cp.wait()


def kernel(x, table):
  xt = x.astype(jnp.int32).T
  tt = table.T
  tail = table[FULL_BLK * 128:]
  tailw = jnp.concatenate([tail, tail], axis=1)
  tw = _convert(tt, tailw)
  o5 = _lookup(tw, xt)
  return o5.transpose(2, 4, 0, 1, 3).reshape(4096, 200, 64)


# speedup vs baseline: 3.9835x; 2.6904x over previous
import functools
import math

import jax
import jax.numpy as jnp
from jax import lax
from jax.experimental import pallas as pl
from jax.experimental.pallas import tpu as pltpu
from jax.experimental.pallas import tpu_sc as plsc

VOCAB = 1_000_000
D_MODEL = 64
SCALE = math.sqrt(D_MODEL)

NUM_CORES = 2
NUM_SUBCORES = 16
LANES = 16
NUM_WORKERS = NUM_CORES * NUM_SUBCORES

CHUNK = 128
NBUF = 4


def _make_lookup(batch_flat: int):
  assert batch_flat % (NUM_WORKERS * CHUNK * NBUF) == 0
  per_worker = batch_flat // NUM_WORKERS
  n_chunks = per_worker // CHUNK

  mesh = plsc.VectorSubcoreMesh(
      core_axis_name="c", subcore_axis_name="s",
      num_cores=NUM_CORES, num_subcores=NUM_SUBCORES)

  @functools.partial(
      pl.kernel,
      out_type=jax.ShapeDtypeStruct((batch_flat, 128), jnp.float32),
      mesh=mesh,
      scratch_types=[
          pltpu.VMEM((n_chunks, CHUNK), jnp.int32),
          pltpu.VMEM((NBUF, CHUNK, 128), jnp.float32),
          pltpu.SemaphoreType.DMA((NBUF,)),
          pltpu.SemaphoreType.DMA((NBUF,)),
      ],
      compiler_params=pltpu.CompilerParams(use_tc_tiling_on_sc=True,
                                           needs_layout_passes=False),
  )
  def lookup(table_hbm, idx_hbm, out_hbm, idx_all, rows, gsem, ssem):
    wid = lax.axis_index("s") * NUM_CORES + lax.axis_index("c")
    base = wid * per_worker
    pltpu.sync_copy(idx_hbm.at[pl.ds(wid * n_chunks, n_chunks)], idx_all)

    @pl.loop(0, n_chunks, step=NBUF)
    def _group(g0):
      gathers = [
          pltpu.async_copy(
              table_hbm.at[idx_all.at[g0 + b]], rows.at[b], gsem.at[b])
          for b in range(NBUF)
      ]
      stores = []
      for b in range(NBUF):
        gathers[b].wait()
        row_buf = rows.at[b]

        @plsc.parallel_loop(0, CHUNK, unroll=4)
        def _scale(r, row_buf=row_buf):
          for j in range(D_MODEL // LANES):
            sl = (r, pl.ds(j * LANES, LANES))
            row_buf[sl] = row_buf[sl] * SCALE

        stores.append(
            pltpu.async_copy(
                row_buf, out_hbm.at[pl.ds(base + (g0 + b) * CHUNK, CHUNK)],
                ssem.at[b]))
      for st in stores:
        st.wait()

  return lookup


def kernel(x, table):
  batch_shape = x.shape
  x_flat = x.reshape(-1).astype(jnp.int32)
  idx2d = x_flat.reshape(-1, CHUNK)
  twide = jnp.pad(table, ((0, 0), (0, 128 - D_MODEL)))
  out = _make_lookup(x_flat.shape[0])(twide, idx2d)
  out3 = out.reshape(*batch_shape, 128)
  return out3[..., :D_MODEL]
